# Initial kernel scaffold; baseline (speedup 1.0000x reference)
import jax
import jax.numpy as jnp
from jax.experimental import pallas as pl


def kernel(pillar_features, voxel_coords):
    raise NotImplementedError("write your pallas kernel here")


# speedup vs baseline: 7.4454x; 7.4454x over previous
import functools

import jax
import jax.numpy as jnp
from jax import lax
from jax.experimental import pallas as pl
from jax.experimental.pallas import tpu as pltpu
from jax.experimental.pallas import tpu_sc as plsc

_B = 4
_C = 64
_NY = 512
_NX = 512
_S = _NY * _NX
_N = _B * _S
_P = _B * 20000

_CH = 128
_NCHUNK = _P // _CH
_NW = 32
_ZB = 16384

_SB = 2048


def _sc_scatter_body(feat_hbm, idx_hbm, canvas_hbm, mask_hbm,
                     idx_v, rows_v, ones_v, zeros_v, sem):
    core = lax.axis_index("c")
    sub = lax.axis_index("s")
    wid = sub * 2 + core

    def _zfill(i, _):
        zeros_v[pl.ds(i * 16, 16)] = jnp.zeros((16,), jnp.int32)
        return ()
    lax.fori_loop(0, _ZB // 16, _zfill, ())

    @pl.when(core == 0)
    def _zero_mask():
        base = sub * (_N // 16)
        for j in range(_N // 16 // _ZB):
            pltpu.sync_copy(zeros_v, mask_hbm.at[pl.ds(base + j * _ZB, _ZB)])

    plsc.subcore_barrier()

    nk = (_NCHUNK + _NW - 1) // _NW
    for k in range(nk):
        cid = wid + _NW * k

        @pl.when(cid < _NCHUNK)
        def _scatter_rows():
            pltpu.sync_copy(idx_hbm.at[pl.ds(cid * _CH, _CH)], idx_v)
            pltpu.sync_copy(feat_hbm.at[pl.ds(cid * _CH, _CH), :], rows_v)
            pltpu.async_copy(rows_v, canvas_hbm.at[idx_v], sem).wait()

    ones_v[...] = jnp.ones((_CH,), jnp.int32)
    nk0 = (_NCHUNK + 15) // 16
    for k in range(nk0):
        cid = sub + 16 * k

        @pl.when((core == 0) & (cid < _NCHUNK))
        def _scatter_ones():
            pltpu.sync_copy(idx_hbm.at[pl.ds(cid * _CH, _CH)], idx_v)
            pltpu.async_copy(ones_v, mask_hbm.at[idx_v], sem).wait()


_sc_scatter = functools.partial(
    pl.kernel,
    out_type=(
        jax.ShapeDtypeStruct((_N, _C), jnp.float32),
        jax.ShapeDtypeStruct((_N,), jnp.int32),
    ),
    mesh=plsc.VectorSubcoreMesh(core_axis_name="c", subcore_axis_name="s"),
    compiler_params=pltpu.CompilerParams(use_tc_tiling_on_sc=False),
    scratch_types=[
        pltpu.VMEM((_CH,), jnp.int32),
        pltpu.VMEM((_CH, _C), jnp.float32),
        pltpu.VMEM((_CH,), jnp.int32),
        pltpu.VMEM((_ZB,), jnp.int32),
        pltpu.SemaphoreType.DMA,
    ],
)(_sc_scatter_body)


def _tc_transpose_body(canvas_ref, mask_ref, out_ref):
    x = canvas_ref[0]
    m = mask_ref[0, 0, 0]
    xt = x.T
    out_ref[0] = jnp.where((m != 0)[None, :], xt, jnp.float32(0))


def _tc_transpose(canvas, mask):
    grid = (_B, _S // _SB)
    return pl.pallas_call(
        _tc_transpose_body,
        grid=grid,
        in_specs=[
            pl.BlockSpec((1, _SB, _C), lambda b, s: (b, s, 0)),
            pl.BlockSpec((1, 1, 1, _SB), lambda b, s: (b, s, 0, 0)),
        ],
        out_specs=pl.BlockSpec((1, _C, _SB), lambda b, s: (b, 0, s)),
        out_shape=jax.ShapeDtypeStruct((_B, _C, _S), jnp.float32),
    )(canvas, mask)


def kernel(pillar_features, voxel_coords):
    coords = voxel_coords.astype(jnp.int32)
    flat_idx = coords[:, 0] * _S + coords[:, 2] * _NX + coords[:, 3]
    canvas, mask = _sc_scatter(pillar_features, flat_idx)
    out = _tc_transpose(canvas.reshape(_B, _S, _C),
                        mask.reshape(_B, _S // _SB, 1, _SB))
    return out.reshape(_B, _C, _NY, _NX)
